# baseline (speedup 1.0000x reference)
import jax
import jax.numpy as jnp
from jax import lax
from jax.experimental import pallas as pl
from jax.experimental.pallas import tpu as pltpu
from jax.experimental.pallas import tpu_sc as plsc

VOCAB = 1_000_000
D = 32
NTOK = 819_200

_NC = 2
_NS = 16
_NW = _NC * _NS
_BPW = NTOK // _NW
_C = 256
_NCH = _BPW // _C
_PR = VOCAB // 4


def _emb_body(idx_hbm, table_hbm, out_hbm, idx_v, pidx_v, rows, tbuf,
              gsa, gsb, osa, osb):
    wid = lax.axis_index("s") * _NC + lax.axis_index("c")
    base = wid * _BPW
    pltpu.sync_copy(idx_hbm.at[pl.ds(base, _BPW)], idx_v)

    iota = lax.iota(jnp.int32, 16)
    gsem = [gsa, gsb]
    osem = [osa, osb]

    def make_pidx(h, j):
        @plsc.parallel_loop(0, _C // 16, 1, unroll=2)
        def _(g):
            v = idx_v[pl.ds(j * _C + g * 16, 16)]
            pidx_v[pl.ds(h * _C + g * 16, 16)] = lax.shift_right_logical(v, 2)

    def fire(h, j):
        del j
        for k in range(_C // 128):
            pltpu.async_copy(
                table_hbm.at[pidx_v.at[pl.ds(h * _C + k * 128, 128)]],
                rows.at[h, pl.ds(k * 128, 128)],
                gsem[h],
            )

    def drain_g(h):
        pltpu.make_async_copy(
            table_hbm.at[pl.ds(0, _C)], rows.at[h], gsem[h]
        ).wait()

    def extract(h, j):
        @plsc.parallel_loop(0, _C // 16, 1, unroll=4)
        def _(g):
            u0 = g * 16
            idxv = idx_v[pl.ds(j * _C + u0, 16)]
            qcol = lax.mul(lax.bitwise_and(idxv, 3), 32)
            rowi = iota + u0
            for f in range(D):
                v = plsc.load_gather(rows.at[h], [rowi, qcol + f])
                tbuf[h, f // 8, f % 8, pl.ds(u0, 16)] = v

    def start_o(h, j):
        tok0 = base + j * _C
        for t in range(4):
            pltpu.async_copy(
                tbuf.at[h, t],
                out_hbm.at[pl.ds(8 * t, 8), pl.ds(tok0, _C)],
                osem[h],
            )

    def wait_o(h):
        for t in range(4):
            pltpu.make_async_copy(
                tbuf.at[h, t], out_hbm.at[pl.ds(0, 8), pl.ds(0, _C)], osem[h]
            ).wait()

    make_pidx(0, 0)
    fire(0, 0)

    def body(i2, carry):
        j = i2 * 2
        for h in (0, 1):
            jj = j + h
            def prefetch(jn=jj + 1, hn=1 - h):
                make_pidx(hn, jn)
                fire(hn, jn)
            pl.when(jj + 1 < _NCH)(prefetch)
            drain_g(h)
            pl.when(jj >= 2)(lambda hh=h: wait_o(hh))
            extract(h, jj)
            start_o(h, jj)
        return carry

    lax.fori_loop(0, _NCH // 2, body, 0)
    wait_o(0)
    wait_o(1)


_emb = pl.kernel(
    _emb_body,
    out_type=jax.ShapeDtypeStruct((D, NTOK), jnp.float32),
    mesh=plsc.VectorSubcoreMesh(core_axis_name="c", subcore_axis_name="s"),
    scratch_types=[
        pltpu.VMEM((_BPW,), jnp.int32),
        pltpu.VMEM((2 * _C,), jnp.int32),
        pltpu.VMEM((2, _C, 128), jnp.float32),
        pltpu.VMEM((2, 4, 8, _C), jnp.float32),
        pltpu.SemaphoreType.DMA,
        pltpu.SemaphoreType.DMA,
        pltpu.SemaphoreType.DMA,
        pltpu.SemaphoreType.DMA,
    ],
    compiler_params=pltpu.CompilerParams(
        use_tc_tiling_on_sc=True, needs_layout_passes=False
    ),
)


@jax.jit
def kernel(x, weights):
    w128 = weights.reshape(_PR, 128)
    return _emb(x.astype(jnp.int32), w128).T


# speedup vs baseline: 1.0644x; 1.0644x over previous
import jax
import jax.numpy as jnp
from jax import lax
from jax.experimental import pallas as pl
from jax.experimental.pallas import tpu as pltpu
from jax.experimental.pallas import tpu_sc as plsc

VOCAB = 1_000_000
D = 32
NTOK = 819_200

_NC = 2
_NS = 16
_NW = _NC * _NS
_BPW = NTOK // _NW
_C = 256
_NCH = _BPW // _C
_PR = VOCAB // 4


def _emb_body(idx_hbm, table_hbm, out_hbm, idx_v, pidx_v, rows, tbuf,
              gsa, gsb, osa, osb):
    wid = lax.axis_index("s") * _NC + lax.axis_index("c")
    base = wid * _BPW
    pltpu.sync_copy(idx_hbm.at[pl.ds(base, _BPW)], idx_v)

    iota = lax.iota(jnp.int32, 16)
    gsem = [gsa, gsb]
    osem = [osa, osb]

    def make_pidx(h, j):
        @plsc.parallel_loop(0, _C // 16, 1, unroll=2)
        def _(g):
            v = idx_v[pl.ds(j * _C + g * 16, 16)]
            pidx_v[pl.ds(h * _C + g * 16, 16)] = lax.shift_right_logical(v, 2)

    def fire(h, j):
        del j
        for k in range(_C // 128):
            pltpu.async_copy(
                table_hbm.at[pidx_v.at[pl.ds(h * _C + k * 128, 128)]],
                rows.at[h, pl.ds(k * 128, 128)],
                gsem[h],
            )

    def drain_g(h):
        pltpu.make_async_copy(
            table_hbm.at[pl.ds(0, _C)], rows.at[h], gsem[h]
        ).wait()

    def extract(h, j):
        @plsc.parallel_loop(0, _C // 16, 1, unroll=1)
        def _(g):
            u0 = g * 16
            idxv = idx_v[pl.ds(j * _C + u0, 16)]
            qcol = lax.mul(lax.bitwise_and(idxv, 3), 32)
            rowi = iota + u0
            for f in range(D):
                v = plsc.load_gather(rows.at[h], [rowi, qcol + f])
                tbuf[h, f // 8, f % 8, pl.ds(u0, 16)] = v

    def start_o(h, j):
        tok0 = base + j * _C
        for t in range(4):
            pltpu.async_copy(
                tbuf.at[h, t],
                out_hbm.at[pl.ds(8 * t, 8), pl.ds(tok0, _C)],
                osem[h],
            )

    def wait_o(h):
        for t in range(4):
            pltpu.make_async_copy(
                tbuf.at[h, t], out_hbm.at[pl.ds(0, 8), pl.ds(0, _C)], osem[h]
            ).wait()

    make_pidx(0, 0)
    fire(0, 0)

    def body(i2, carry):
        j = i2 * 2
        for h in (0, 1):
            jj = j + h
            def prefetch(jn=jj + 1, hn=1 - h):
                make_pidx(hn, jn)
                fire(hn, jn)
            pl.when(jj + 1 < _NCH)(prefetch)
            drain_g(h)
            pl.when(jj >= 2)(lambda hh=h: wait_o(hh))
            extract(h, jj)
            start_o(h, jj)
        return carry

    lax.fori_loop(0, _NCH // 2, body, 0)
    wait_o(0)
    wait_o(1)


_emb = pl.kernel(
    _emb_body,
    out_type=jax.ShapeDtypeStruct((D, NTOK), jnp.float32),
    mesh=plsc.VectorSubcoreMesh(core_axis_name="c", subcore_axis_name="s"),
    scratch_types=[
        pltpu.VMEM((_BPW,), jnp.int32),
        pltpu.VMEM((2 * _C,), jnp.int32),
        pltpu.VMEM((2, _C, 128), jnp.float32),
        pltpu.VMEM((2, 4, 8, _C), jnp.float32),
        pltpu.SemaphoreType.DMA,
        pltpu.SemaphoreType.DMA,
        pltpu.SemaphoreType.DMA,
        pltpu.SemaphoreType.DMA,
    ],
    compiler_params=pltpu.CompilerParams(
        use_tc_tiling_on_sc=True, needs_layout_passes=False
    ),
)


@jax.jit
def kernel(x, weights):
    w128 = weights.reshape(_PR, 128)
    return _emb(x.astype(jnp.int32), w128).T


# speedup vs baseline: 1.0648x; 1.0003x over previous
import jax
import jax.numpy as jnp
from jax import lax
from jax.experimental import pallas as pl
from jax.experimental.pallas import tpu as pltpu
from jax.experimental.pallas import tpu_sc as plsc

VOCAB = 1_000_000
D = 32
NTOK = 819_200

_NC = 2
_NS = 16
_NW = _NC * _NS
_BPW = NTOK // _NW
_C = 256
_NCH = _BPW // _C
_PR = VOCAB // 4


def _emb_body(idx_hbm, table_hbm, out_hbm, idx_v, pidx_v, rows, tbuf,
              gsa, gsb, osa, osb):
    wid = lax.axis_index("s") * _NC + lax.axis_index("c")
    base = wid * _BPW
    pltpu.sync_copy(idx_hbm.at[pl.ds(base, _BPW)], idx_v)

    iota = lax.iota(jnp.int32, 16)
    gsem = [gsa, gsb]
    osem = [osa, osb]

    def make_pidx(h, j):
        @plsc.parallel_loop(0, _C // 16, 1, unroll=2)
        def _(g):
            v = idx_v[pl.ds(j * _C + g * 16, 16)]
            pidx_v[pl.ds(h * _C + g * 16, 16)] = lax.shift_right_logical(v, 2)

    def fire(h, j):
        del j
        for k in range(_C // 128):
            pltpu.async_copy(
                table_hbm.at[pidx_v.at[pl.ds(h * _C + k * 128, 128)]],
                rows.at[h, pl.ds(k * 128, 128)],
                gsem[h],
            )

    def drain_g(h):
        pltpu.make_async_copy(
            table_hbm.at[pl.ds(0, _C)], rows.at[h], gsem[h]
        ).wait()

    def extract(h, j):
        @plsc.parallel_loop(0, _C // 16, 1, unroll=1)
        def _(g):
            u0 = g * 16
            idxv = idx_v[pl.ds(j * _C + u0, 16)]
            qcol = lax.mul(lax.bitwise_and(idxv, 3), 32)
            rowi = iota + u0
            for f in range(D):
                v = plsc.load_gather(rows.at[h], [rowi, qcol + f])
                tbuf[h, f // 8, f % 8, pl.ds(u0, 16)] = v

    def start_o(h, j):
        tok0 = base + j * _C
        for t in range(4):
            pltpu.async_copy(
                tbuf.at[h, t],
                out_hbm.at[pl.ds(8 * t, 8), pl.ds(tok0, _C)],
                osem[h],
            )

    def wait_o(h):
        for t in range(4):
            pltpu.make_async_copy(
                tbuf.at[h, t], out_hbm.at[pl.ds(0, 8), pl.ds(0, _C)], osem[h]
            ).wait()

    make_pidx(0, 0)
    fire(0, 0)

    def body(i2, carry):
        j = i2 * 2
        for h in (0, 1):
            jj = j + h
            def prefetch(jn=jj + 1, hn=1 - h):
                make_pidx(hn, jn)
                fire(hn, jn)
            pl.when(jj + 1 < _NCH)(prefetch)
            drain_g(h)
            pl.when(jj >= 2)(lambda hh=h: wait_o(hh))
            extract(h, jj)
            start_o(h, jj)
        return carry

    lax.fori_loop(0, _NCH // 2, body, 0)
    wait_o(0)
    wait_o(1)


_emb = pl.kernel(
    _emb_body,
    out_type=jax.ShapeDtypeStruct((D, NTOK), jnp.float32),
    mesh=plsc.VectorSubcoreMesh(core_axis_name="c", subcore_axis_name="s"),
    scratch_types=[
        pltpu.VMEM((_BPW,), jnp.int32),
        pltpu.VMEM((2 * _C,), jnp.int32),
        pltpu.VMEM((2, _C, 128), jnp.float32),
        pltpu.VMEM((2, 4, 8, _C), jnp.float32),
        pltpu.SemaphoreType.DMA,
        pltpu.SemaphoreType.DMA,
        pltpu.SemaphoreType.DMA,
        pltpu.SemaphoreType.DMA,
    ],
    compiler_params=pltpu.CompilerParams(
        use_tc_tiling_on_sc=True,
        needs_layout_passes=False,
        disable_bounds_checks=True,
    ),
)


@jax.jit
def kernel(x, weights):
    w128 = weights.reshape(_PR, 128)
    return _emb(x.astype(jnp.int32), w128).T


# speedup vs baseline: 1.1638x; 1.0930x over previous
import jax
import jax.numpy as jnp
from jax import lax
from jax.experimental import pallas as pl
from jax.experimental.pallas import tpu as pltpu
from jax.experimental.pallas import tpu_sc as plsc

VOCAB = 1_000_000
D = 32
NTOK = 819_200

_NC = 2
_NS = 16
_NW = _NC * _NS
_BPW = NTOK // _NW
_C = 256
_NCH = _BPW // _C
_PR = VOCAB // 4
_TC = VOCAB // 128
_CPW = (_TC + 1 + _NW - 1) // _NW

_params = pltpu.CompilerParams(
    use_tc_tiling_on_sc=True,
    needs_layout_passes=False,
    disable_bounds_checks=True,
)


def _tpose_body(wt_hbm, w128_hbm, vin, vout, isa, isb, osa, osb):
    wid = lax.axis_index("s") * _NC + lax.axis_index("c")
    iota = lax.iota(jnp.int32, 16)
    isem = [isa, isb]
    osem = [osa, osb]
    fcv = [lax.bitwise_and(iota, 15) + 16 * odd for odd in range(2)]
    tcv = [lax.shift_right_logical(f, 3) for f in fcv]
    frv = [lax.bitwise_and(f, 7) for f in fcv]

    def fire(b, cc, w):
        v0 = cc * 128
        for t in range(4):
            pltpu.async_copy(
                wt_hbm.at[pl.ds(8 * t, 8), pl.ds(v0, w)],
                vin.at[b, t, slice(None), pl.ds(0, w)],
                isem[b],
            )

    def drain_in(b, w):
        for t in range(4):
            pltpu.make_async_copy(
                wt_hbm.at[pl.ds(0, 8), pl.ds(0, w)],
                vin.at[b, t, slice(None), pl.ds(0, w)],
                isem[b],
            ).wait()

    def shuffle(b, npr):
        @plsc.parallel_loop(0, npr, 1, unroll=1)
        def _(pr):
            for c in range(8):
                us = jnp.full((16,), pr * 4 + (c >> 1), jnp.int32)
                v = plsc.load_gather(vin.at[b], [tcv[c & 1], frv[c & 1], us])
                vout[b, pr, pl.ds(c * 16, 16)] = v

    def start_o(b, cc, npr):
        pltpu.async_copy(
            vout.at[b, pl.ds(0, npr)],
            w128_hbm.at[pl.ds(cc * 32, npr)],
            osem[b],
        )

    def wait_o(b, npr):
        pltpu.make_async_copy(
            vout.at[b, pl.ds(0, npr)], w128_hbm.at[pl.ds(0, npr)], osem[b]
        ).wait()

    def cc_of(ci):
        return wid + ci * _NW

    def process(b, ci):
        cc = cc_of(ci)
        full = cc < _TC

        def go(w, npr):
            drain_in(b, w)
            pl.when(ci >= 2)(lambda: wait_o(b, 32))
            shuffle(b, npr)
            start_o(b, cc, npr)

        pl.when(full)(lambda: go(128, 32))
        pl.when(cc == _TC)(lambda: go(64, 16))
        cn = cc_of(ci + 2)
        pl.when(cn < _TC)(lambda: fire(b, cn, 128))
        pl.when(cn == _TC)(lambda: fire(b, cn, 64))

    fire(0, cc_of(0), 128)
    fire(1, cc_of(1), 128)

    def body(i2, carry):
        process(0, i2 * 2)
        process(1, i2 * 2 + 1)
        return carry

    lax.fori_loop(0, (_CPW + 1) // 2, body, 0)
    tail_ci = _TC - wid
    has_tail = lax.rem(tail_ci, _NW) == 0
    pl.when(has_tail)(lambda: wait_o(0, 16))
    pl.when(jnp.logical_not(has_tail))(lambda: wait_o(0, 32))
    wait_o(1, 32)


_tpose = pl.kernel(
    _tpose_body,
    out_type=jax.ShapeDtypeStruct((_PR, 128), jnp.float32),
    mesh=plsc.VectorSubcoreMesh(core_axis_name="c", subcore_axis_name="s"),
    scratch_types=[
        pltpu.VMEM((2, 4, 8, 128), jnp.float32),
        pltpu.VMEM((2, 32, 128), jnp.float32),
        pltpu.SemaphoreType.DMA,
        pltpu.SemaphoreType.DMA,
        pltpu.SemaphoreType.DMA,
        pltpu.SemaphoreType.DMA,
    ],
    compiler_params=_params,
)


def _emb_body(idx_hbm, table_hbm, out_hbm, idx_v, pidx_v, rows, tbuf,
              gsa, gsb, osa, osb):
    wid = lax.axis_index("s") * _NC + lax.axis_index("c")
    base = wid * _BPW
    pltpu.sync_copy(idx_hbm.at[pl.ds(base, _BPW)], idx_v)

    iota = lax.iota(jnp.int32, 16)
    gsem = [gsa, gsb]
    osem = [osa, osb]

    def make_pidx(h, j):
        @plsc.parallel_loop(0, _C // 16, 1, unroll=2)
        def _(g):
            v = idx_v[pl.ds(j * _C + g * 16, 16)]
            pidx_v[pl.ds(h * _C + g * 16, 16)] = lax.shift_right_logical(v, 2)

    def fire(h, j):
        del j
        for k in range(_C // 128):
            pltpu.async_copy(
                table_hbm.at[pidx_v.at[pl.ds(h * _C + k * 128, 128)]],
                rows.at[h, pl.ds(k * 128, 128)],
                gsem[h],
            )

    def drain_g(h):
        pltpu.make_async_copy(
            table_hbm.at[pl.ds(0, _C)], rows.at[h], gsem[h]
        ).wait()

    def extract(h, j):
        @plsc.parallel_loop(0, _C // 16, 1, unroll=1)
        def _(g):
            u0 = g * 16
            idxv = idx_v[pl.ds(j * _C + u0, 16)]
            qcol = lax.mul(lax.bitwise_and(idxv, 3), 32)
            rowi = iota + u0
            for f in range(D):
                v = plsc.load_gather(rows.at[h], [rowi, qcol + f])
                tbuf[h, f // 8, f % 8, pl.ds(u0, 16)] = v

    def start_o(h, j):
        tok0 = base + j * _C
        for t in range(4):
            pltpu.async_copy(
                tbuf.at[h, t],
                out_hbm.at[pl.ds(8 * t, 8), pl.ds(tok0, _C)],
                osem[h],
            )

    def wait_o(h):
        for t in range(4):
            pltpu.make_async_copy(
                tbuf.at[h, t], out_hbm.at[pl.ds(0, 8), pl.ds(0, _C)], osem[h]
            ).wait()

    make_pidx(0, 0)
    fire(0, 0)

    def body(i2, carry):
        j = i2 * 2
        for h in (0, 1):
            jj = j + h

            def prefetch(jn=jj + 1, hn=1 - h):
                make_pidx(hn, jn)
                fire(hn, jn)

            pl.when(jj + 1 < _NCH)(prefetch)
            drain_g(h)
            pl.when(jj >= 2)(lambda hh=h: wait_o(hh))
            extract(h, jj)
            start_o(h, jj)
        return carry

    lax.fori_loop(0, _NCH // 2, body, 0)
    wait_o(0)
    wait_o(1)


_emb = pl.kernel(
    _emb_body,
    out_type=jax.ShapeDtypeStruct((D, NTOK), jnp.float32),
    mesh=plsc.VectorSubcoreMesh(core_axis_name="c", subcore_axis_name="s"),
    scratch_types=[
        pltpu.VMEM((_BPW,), jnp.int32),
        pltpu.VMEM((2 * _C,), jnp.int32),
        pltpu.VMEM((2, _C, 128), jnp.float32),
        pltpu.VMEM((2, 4, 8, _C), jnp.float32),
        pltpu.SemaphoreType.DMA,
        pltpu.SemaphoreType.DMA,
        pltpu.SemaphoreType.DMA,
        pltpu.SemaphoreType.DMA,
    ],
    compiler_params=_params,
)


@jax.jit
def kernel(x, weights):
    w128 = _tpose(weights.T)
    return _emb(x.astype(jnp.int32), w128).T


# speedup vs baseline: 1.1693x; 1.0048x over previous
import jax
import jax.numpy as jnp
from jax import lax
from jax.experimental import pallas as pl
from jax.experimental.pallas import tpu as pltpu
from jax.experimental.pallas import tpu_sc as plsc

VOCAB = 1_000_000
D = 32
NTOK = 819_200

_NC = 2
_NS = 16
_NW = _NC * _NS
_BPW = NTOK // _NW
_C = 256
_NCH = _BPW // _C
_PR = VOCAB // 4
_TC = VOCAB // 128
_CPW = (_TC + 1 + _NW - 1) // _NW

_params = pltpu.CompilerParams(
    use_tc_tiling_on_sc=True,
    needs_layout_passes=False,
    disable_bounds_checks=True,
)


def _tpose_body(wt_hbm, w128_hbm, vin, vout, isa, isb, osa, osb):
    wid = lax.axis_index("s") * _NC + lax.axis_index("c")
    iota = lax.iota(jnp.int32, 16)
    isem = [isa, isb]
    osem = [osa, osb]
    fcv = [lax.bitwise_and(iota, 15) + 16 * odd for odd in range(2)]
    tcv = [lax.shift_right_logical(f, 3) for f in fcv]
    frv = [lax.bitwise_and(f, 7) for f in fcv]

    def fire(b, cc, w):
        v0 = cc * 128
        for t in range(4):
            pltpu.async_copy(
                wt_hbm.at[pl.ds(8 * t, 8), pl.ds(v0, w)],
                vin.at[b, t, slice(None), pl.ds(0, w)],
                isem[b],
            )

    def drain_in(b, w):
        for t in range(4):
            pltpu.make_async_copy(
                wt_hbm.at[pl.ds(0, 8), pl.ds(0, w)],
                vin.at[b, t, slice(None), pl.ds(0, w)],
                isem[b],
            ).wait()

    def shuffle(b, npr):
        @plsc.parallel_loop(0, npr, 1, unroll=4)
        def _(pr):
            for c in range(8):
                us = jnp.full((16,), pr * 4 + (c >> 1), jnp.int32)
                v = plsc.load_gather(vin.at[b], [tcv[c & 1], frv[c & 1], us])
                vout[b, pr, pl.ds(c * 16, 16)] = v

    def start_o(b, cc, npr):
        pltpu.async_copy(
            vout.at[b, pl.ds(0, npr)],
            w128_hbm.at[pl.ds(cc * 32, npr)],
            osem[b],
        )

    def wait_o(b, npr):
        pltpu.make_async_copy(
            vout.at[b, pl.ds(0, npr)], w128_hbm.at[pl.ds(0, npr)], osem[b]
        ).wait()

    def cc_of(ci):
        return wid + ci * _NW

    def process(b, ci):
        cc = cc_of(ci)
        full = cc < _TC

        def go(w, npr):
            drain_in(b, w)
            pl.when(ci >= 2)(lambda: wait_o(b, 32))
            shuffle(b, npr)
            start_o(b, cc, npr)

        pl.when(full)(lambda: go(128, 32))
        pl.when(cc == _TC)(lambda: go(64, 16))
        cn = cc_of(ci + 2)
        pl.when(cn < _TC)(lambda: fire(b, cn, 128))
        pl.when(cn == _TC)(lambda: fire(b, cn, 64))

    fire(0, cc_of(0), 128)
    fire(1, cc_of(1), 128)

    def body(i2, carry):
        process(0, i2 * 2)
        process(1, i2 * 2 + 1)
        return carry

    lax.fori_loop(0, (_CPW + 1) // 2, body, 0)
    tail_ci = _TC - wid
    has_tail = lax.rem(tail_ci, _NW) == 0
    pl.when(has_tail)(lambda: wait_o(0, 16))
    pl.when(jnp.logical_not(has_tail))(lambda: wait_o(0, 32))
    wait_o(1, 32)


_tpose = pl.kernel(
    _tpose_body,
    out_type=jax.ShapeDtypeStruct((_PR, 128), jnp.float32),
    mesh=plsc.VectorSubcoreMesh(core_axis_name="c", subcore_axis_name="s"),
    scratch_types=[
        pltpu.VMEM((2, 4, 8, 128), jnp.float32),
        pltpu.VMEM((2, 32, 128), jnp.float32),
        pltpu.SemaphoreType.DMA,
        pltpu.SemaphoreType.DMA,
        pltpu.SemaphoreType.DMA,
        pltpu.SemaphoreType.DMA,
    ],
    compiler_params=_params,
)


def _emb_body(idx_hbm, table_hbm, out_hbm, idx_v, pidx_v, rows, tbuf,
              gsa, gsb, osa, osb):
    wid = lax.axis_index("s") * _NC + lax.axis_index("c")
    base = wid * _BPW
    pltpu.sync_copy(idx_hbm.at[pl.ds(base, _BPW)], idx_v)

    iota = lax.iota(jnp.int32, 16)
    gsem = [gsa, gsb]
    osem = [osa, osb]

    def make_pidx(h, j):
        @plsc.parallel_loop(0, _C // 16, 1, unroll=2)
        def _(g):
            v = idx_v[pl.ds(j * _C + g * 16, 16)]
            pidx_v[pl.ds(h * _C + g * 16, 16)] = lax.shift_right_logical(v, 2)

    def fire(h, j):
        del j
        for k in range(_C // 128):
            pltpu.async_copy(
                table_hbm.at[pidx_v.at[pl.ds(h * _C + k * 128, 128)]],
                rows.at[h, pl.ds(k * 128, 128)],
                gsem[h],
            )

    def drain_g(h):
        pltpu.make_async_copy(
            table_hbm.at[pl.ds(0, _C)], rows.at[h], gsem[h]
        ).wait()

    def extract(h, j):
        @plsc.parallel_loop(0, _C // 16, 1, unroll=1)
        def _(g):
            u0 = g * 16
            idxv = idx_v[pl.ds(j * _C + u0, 16)]
            qcol = lax.mul(lax.bitwise_and(idxv, 3), 32)
            rowi = iota + u0
            for f in range(D):
                v = plsc.load_gather(rows.at[h], [rowi, qcol + f])
                tbuf[h, f // 8, f % 8, pl.ds(u0, 16)] = v

    def start_o(h, j):
        tok0 = base + j * _C
        for t in range(4):
            pltpu.async_copy(
                tbuf.at[h, t],
                out_hbm.at[pl.ds(8 * t, 8), pl.ds(tok0, _C)],
                osem[h],
            )

    def wait_o(h):
        for t in range(4):
            pltpu.make_async_copy(
                tbuf.at[h, t], out_hbm.at[pl.ds(0, 8), pl.ds(0, _C)], osem[h]
            ).wait()

    make_pidx(0, 0)
    fire(0, 0)

    def body(i2, carry):
        j = i2 * 2
        for h in (0, 1):
            jj = j + h

            def prefetch(jn=jj + 1, hn=1 - h):
                make_pidx(hn, jn)
                fire(hn, jn)

            pl.when(jj + 1 < _NCH)(prefetch)
            drain_g(h)
            pl.when(jj >= 2)(lambda hh=h: wait_o(hh))
            extract(h, jj)
            start_o(h, jj)
        return carry

    lax.fori_loop(0, _NCH // 2, body, 0)
    wait_o(0)
    wait_o(1)


_emb = pl.kernel(
    _emb_body,
    out_type=jax.ShapeDtypeStruct((D, NTOK), jnp.float32),
    mesh=plsc.VectorSubcoreMesh(core_axis_name="c", subcore_axis_name="s"),
    scratch_types=[
        pltpu.VMEM((_BPW,), jnp.int32),
        pltpu.VMEM((2 * _C,), jnp.int32),
        pltpu.VMEM((2, _C, 128), jnp.float32),
        pltpu.VMEM((2, 4, 8, _C), jnp.float32),
        pltpu.SemaphoreType.DMA,
        pltpu.SemaphoreType.DMA,
        pltpu.SemaphoreType.DMA,
        pltpu.SemaphoreType.DMA,
    ],
    compiler_params=_params,
)


@jax.jit
def kernel(x, weights):
    w128 = _tpose(weights.T)
    return _emb(x.astype(jnp.int32), w128).T
